# baseline (speedup 1.0000x reference)
import functools

import jax
import jax.numpy as jnp
from jax import lax
from jax.experimental import pallas as pl
from jax.experimental.pallas import tpu as pltpu
from jax.experimental.pallas import tpu_sc as plsc

_LANES = 16
_NBUF = 3
_NSPLIT = 2
_UNROLL = 8


def _sc_cumsum_2d(x2d, batch, seq):
    rows, feat = x2d.shape
    info = plsc.get_sparse_core_info()
    nc, ns = info.num_cores, info.num_subcores
    nw = nc * ns
    strips_per_batch = nw // batch
    fw = feat // strips_per_batch
    assert feat % strips_per_batch == 0 and fw % _LANES == 0
    nvec = fw // _LANES
    ch = 256
    assert seq % ch == 0
    nchunks = seq // ch

    mesh = plsc.VectorSubcoreMesh(core_axis_name="c", subcore_axis_name="s")

    @functools.partial(
        pl.kernel,
        mesh=mesh,
        out_type=jax.ShapeDtypeStruct((rows, feat), jnp.float32),
        scratch_types=(
            [pltpu.VMEM((ch, fw), jnp.float32) for _ in range(_NBUF)]
            + [pltpu.SemaphoreType.DMA for _ in range(2 * _NBUF * _NSPLIT)]
        ),
    )
    def run(x_hbm, o_hbm, *scratch):
        bufs = scratch[:_NBUF]
        sems = scratch[_NBUF:]
        in_sems = [
            sems[i * _NSPLIT : (i + 1) * _NSPLIT] for i in range(_NBUF)
        ]
        out_sems = [
            sems[(_NBUF + i) * _NSPLIT : (_NBUF + i + 1) * _NSPLIT]
            for i in range(_NBUF)
        ]
        part = ch // _NSPLIT

        wid = lax.axis_index("s") * nc + lax.axis_index("c")
        b = wid // strips_per_batch
        f0 = pl.multiple_of((wid % strips_per_batch) * fw, fw)
        row0 = b * seq

        def copy_in(ci, p):
            return [
                pltpu.async_copy(
                    x_hbm.at[pl.ds(row0 + ci * ch + k * part, part), pl.ds(f0, fw)],
                    bufs[p].at[pl.ds(k * part, part), :],
                    in_sems[p][k],
                )
                for k in range(_NSPLIT)
            ]

        def copy_out(ci, p):
            return [
                pltpu.async_copy(
                    bufs[p].at[pl.ds(k * part, part), :],
                    o_hbm.at[pl.ds(row0 + ci * ch + k * part, part), pl.ds(f0, fw)],
                    out_sems[p][k],
                )
                for k in range(_NSPLIT)
            ]

        def make_row_body(p):
            def row_body(t, accs):
                new = accs
                for u in range(_UNROLL):
                    cur = []
                    for j in range(nvec):
                        a = new[j] + bufs[p][t * _UNROLL + u, pl.ds(j * _LANES, _LANES)]
                        bufs[p][t * _UNROLL + u, pl.ds(j * _LANES, _LANES)] = a
                        cur.append(a)
                    new = tuple(cur)
                return new

            return row_body

        in_handles = [None] * nchunks
        out_handles = [None] * nchunks
        in_handles[0] = copy_in(0, 0)
        accs = tuple(jnp.zeros((_LANES,), jnp.float32) for _ in range(nvec))
        for ci in range(nchunks):
            p = ci % _NBUF
            if ci + 1 < nchunks:
                q = (ci + 1) % _NBUF
                if ci - 2 >= 0:
                    for h in out_handles[ci - 2]:
                        h.wait()
                in_handles[ci + 1] = copy_in(ci + 1, q)
            for h in in_handles[ci]:
                h.wait()
            accs = lax.fori_loop(0, ch // _UNROLL, make_row_body(p), accs)
            out_handles[ci] = copy_out(ci, p)
        for ci in (nchunks - 2, nchunks - 1):
            for h in out_handles[ci]:
                h.wait()

    return run(x2d)


def kernel(x, dim):
    del dim
    b, s, f = x.shape
    out = _sc_cumsum_2d(x.reshape(b * s, f), b, s)
    return out.reshape(b, s, f)


# speedup vs baseline: 1.0366x; 1.0366x over previous
import functools

import jax
import jax.numpy as jnp
from jax import lax
from jax.experimental import pallas as pl
from jax.experimental.pallas import tpu as pltpu
from jax.experimental.pallas import tpu_sc as plsc

_LANES = 16
_NBUF = 3
_NSPLIT = 2
_UNROLL = 8


def _sc_cumsum_2d(x2d, batch, seq):
    rows, feat = x2d.shape
    info = plsc.get_sparse_core_info()
    nc, ns = info.num_cores, info.num_subcores
    nw = nc * ns
    strips_per_batch = nw // batch
    fw = feat // strips_per_batch
    assert feat % strips_per_batch == 0 and fw % _LANES == 0
    nvec = fw // _LANES
    ch = 256
    assert seq % ch == 0
    nchunks = seq // ch

    mesh = plsc.VectorSubcoreMesh(core_axis_name="c", subcore_axis_name="s")

    @functools.partial(
        pl.kernel,
        mesh=mesh,
        out_type=jax.ShapeDtypeStruct((rows, feat), jnp.float32),
        scratch_types=(
            [pltpu.VMEM((ch, fw), jnp.float32) for _ in range(_NBUF)]
            + [pltpu.SemaphoreType.DMA for _ in range(2 * _NBUF * _NSPLIT)]
        ),
    )
    def run(x_hbm, o_hbm, *scratch):
        bufs = scratch[:_NBUF]
        sems = scratch[_NBUF:]
        in_sems = [
            sems[i * _NSPLIT : (i + 1) * _NSPLIT] for i in range(_NBUF)
        ]
        out_sems = [
            sems[(_NBUF + i) * _NSPLIT : (_NBUF + i + 1) * _NSPLIT]
            for i in range(_NBUF)
        ]
        part = ch // _NSPLIT

        wid = lax.axis_index("s") * nc + lax.axis_index("c")
        b = wid // strips_per_batch
        f0 = pl.multiple_of((wid % strips_per_batch) * fw, fw)
        row0 = b * seq

        def copy_in(ci, p):
            return [
                pltpu.async_copy(
                    x_hbm.at[pl.ds(row0 + ci * ch + k * part, part), pl.ds(f0, fw)],
                    bufs[p].at[pl.ds(k * part, part), :],
                    in_sems[p][k],
                )
                for k in range(_NSPLIT)
            ]

        def copy_out(ci, p):
            return [
                pltpu.async_copy(
                    bufs[p].at[pl.ds(k * part, part), :],
                    o_hbm.at[pl.ds(row0 + ci * ch + k * part, part), pl.ds(f0, fw)],
                    out_sems[p][k],
                )
                for k in range(_NSPLIT)
            ]

        def scan_chunk(p, accs):
            @plsc.parallel_loop(0, ch, unroll=_UNROLL, carry=accs)
            def row_body(t, accs):
                cur = []
                for j in range(nvec):
                    a = accs[j] + bufs[p][t, pl.ds(j * _LANES, _LANES)]
                    bufs[p][t, pl.ds(j * _LANES, _LANES)] = a
                    cur.append(a)
                return tuple(cur)

            return row_body

        in_handles = [None] * nchunks
        out_handles = [None] * nchunks
        in_handles[0] = copy_in(0, 0)
        accs = tuple(jnp.zeros((_LANES,), jnp.float32) for _ in range(nvec))
        for ci in range(nchunks):
            p = ci % _NBUF
            if ci + 1 < nchunks:
                q = (ci + 1) % _NBUF
                if ci - 2 >= 0:
                    for h in out_handles[ci - 2]:
                        h.wait()
                in_handles[ci + 1] = copy_in(ci + 1, q)
            for h in in_handles[ci]:
                h.wait()
            accs = scan_chunk(p, accs)
            out_handles[ci] = copy_out(ci, p)
        for ci in (nchunks - 2, nchunks - 1):
            for h in out_handles[ci]:
                h.wait()

    return run(x2d)


def kernel(x, dim):
    del dim
    b, s, f = x.shape
    out = _sc_cumsum_2d(x.reshape(b * s, f), b, s)
    return out.reshape(b, s, f)
